# baseline (speedup 1.0000x reference)
import functools

import jax
import jax.numpy as jnp
from jax import lax
from jax.experimental import pallas as pl
from jax.experimental.pallas import tpu as pltpu
from jax.experimental.pallas import tpu_sc as plsc

_LANES = 16
_NUM_WORKERS = 32
_CHUNK = 16384
_NBUF = 2
_UNROLL = 4


def _sc_kernel_body(n_thr, per_worker, scores_hbm, thr_hbm, out_hbm,
                    thr_v, in0, in1, out0, out1,
                    si0, si1, so0, so1):
    in_b = (in0, in1)
    out_b = (out0, out1)
    in_sem = (si0, si1)
    out_sem = (so0, so1)

    wid = lax.axis_index("s") * 2 + lax.axis_index("c")
    base_off = wid * per_worker
    num_chunks = per_worker // _CHUNK

    pltpu.sync_copy(thr_hbm, thr_v)
    t7b = thr_v[pl.ds(_LANES, _LANES)]

    for b in range(_NBUF):
        pltpu.async_copy(
            scores_hbm.at[pl.ds(base_off + b * _CHUNK, _CHUNK)],
            in_b[b], in_sem[b])

    @pl.loop(0, num_chunks, step=_NBUF)
    def _chunks(g0):
        for b in range(_NBUF):
            g = g0 + b
            off = base_off + g * _CHUNK
            pltpu.make_async_copy(
                scores_hbm.at[pl.ds(off, _CHUNK)], in_b[b], in_sem[b]).wait()

            @pl.when(g >= _NBUF)
            def _():
                pltpu.make_async_copy(
                    out_b[b],
                    out_hbm.at[pl.ds(off - _NBUF * _CHUNK, _CHUNK)],
                    out_sem[b]).wait()

            @pl.loop(0, _CHUNK // (_LANES * _UNROLL))
            def _vecs(i):
                for u in range(_UNROLL):
                    s = (i * _UNROLL + u) * _LANES
                    v = in_b[b][pl.ds(s, _LANES)]
                    base = jnp.where(v > t7b, 8, 0)
                    g = plsc.load_gather(thr_v, [base + 3])
                    base = base + jnp.where(v > g, 4, 0)
                    g = plsc.load_gather(thr_v, [base + 1])
                    base = base + jnp.where(v > g, 2, 0)
                    g = plsc.load_gather(thr_v, [base])
                    base = base + jnp.where(v > g, 1, 0)
                    out_b[b][pl.ds(s, _LANES)] = base

            pltpu.async_copy(out_b[b], out_hbm.at[pl.ds(off, _CHUNK)],
                             out_sem[b])

            @pl.when(g + _NBUF < num_chunks)
            def _():
                pltpu.async_copy(
                    scores_hbm.at[pl.ds(off + _NBUF * _CHUNK, _CHUNK)],
                    in_b[b], in_sem[b])

    for b in range(_NBUF):
        last_off = base_off + (num_chunks - _NBUF + b) * _CHUNK
        pltpu.make_async_copy(
            out_b[b], out_hbm.at[pl.ds(last_off, _CHUNK)], out_sem[b]).wait()


def kernel(scores, thresholds):
    n = scores.shape[0]
    n_thr = thresholds.shape[0]
    assert n % (_NUM_WORKERS * _CHUNK * _NBUF) == 0
    per_worker = n // _NUM_WORKERS

    thr_f = thresholds.astype(jnp.float32)
    thr_b = jnp.concatenate([
        thr_f, jnp.full((_LANES - n_thr,), jnp.inf, jnp.float32),
        jnp.full((_LANES,), thr_f[7], jnp.float32),
    ])

    mesh = plsc.VectorSubcoreMesh(core_axis_name="c", subcore_axis_name="s")
    fn = functools.partial(
        pl.kernel,
        out_type=jax.ShapeDtypeStruct((n,), jnp.int32),
        mesh=mesh,
        compiler_params=pltpu.CompilerParams(needs_layout_passes=False),
        scratch_types=[
            pltpu.VMEM((2 * _LANES,), jnp.float32),
            pltpu.VMEM((_CHUNK,), jnp.float32),
            pltpu.VMEM((_CHUNK,), jnp.float32),
            pltpu.VMEM((_CHUNK,), jnp.int32),
            pltpu.VMEM((_CHUNK,), jnp.int32),
            pltpu.SemaphoreType.DMA,
            pltpu.SemaphoreType.DMA,
            pltpu.SemaphoreType.DMA,
            pltpu.SemaphoreType.DMA,
        ],
    )(functools.partial(_sc_kernel_body, n_thr, per_worker))
    return fn(scores, thr_b)


# speedup vs baseline: 3.2019x; 3.2019x over previous
import functools

import jax
import jax.numpy as jnp
from jax import lax
from jax.experimental import pallas as pl
from jax.experimental.pallas import tpu as pltpu
from jax.experimental.pallas import tpu_sc as plsc

_LANES = 16
_NUM_WORKERS = 32
_CHUNK = 16384
_NBUF = 2
_UNROLL = 4


def _sc_kernel_body(n_thr, per_worker, scores_hbm, thr_hbm, out_hbm,
                    thr_v, in0, in1, out0, out1,
                    si0, si1, so0, so1):
    in_b = (in0, in1)
    out_b = (out0, out1)
    in_sem = (si0, si1)
    out_sem = (so0, so1)

    wid = lax.axis_index("s") * 2 + lax.axis_index("c")
    base_off = wid * per_worker
    num_chunks = per_worker // _CHUNK

    pltpu.sync_copy(thr_hbm, thr_v)
    tb = [thr_v[pl.ds(j * _LANES, _LANES)] for j in range(n_thr)]

    for b in range(_NBUF):
        pltpu.async_copy(
            scores_hbm.at[pl.ds(base_off + b * _CHUNK, _CHUNK)],
            in_b[b], in_sem[b])

    @pl.loop(0, num_chunks, step=_NBUF)
    def _chunks(g0):
        for b in range(_NBUF):
            g = g0 + b
            off = base_off + g * _CHUNK
            pltpu.make_async_copy(
                scores_hbm.at[pl.ds(off, _CHUNK)], in_b[b], in_sem[b]).wait()

            @pl.when(g >= _NBUF)
            def _():
                pltpu.make_async_copy(
                    out_b[b],
                    out_hbm.at[pl.ds(off - _NBUF * _CHUNK, _CHUNK)],
                    out_sem[b]).wait()

            @pl.loop(0, _CHUNK // (_LANES * _UNROLL))
            def _vecs(i):
                for u in range(_UNROLL):
                    s = (i * _UNROLL + u) * _LANES
                    v = in_b[b][pl.ds(s, _LANES)]
                    acc = jnp.zeros((_LANES,), jnp.int32)
                    for j in range(n_thr):
                        acc = acc + jnp.where(v > tb[j], 1, 0)
                    out_b[b][pl.ds(s, _LANES)] = acc

            pltpu.async_copy(out_b[b], out_hbm.at[pl.ds(off, _CHUNK)],
                             out_sem[b])

            @pl.when(g + _NBUF < num_chunks)
            def _():
                pltpu.async_copy(
                    scores_hbm.at[pl.ds(off + _NBUF * _CHUNK, _CHUNK)],
                    in_b[b], in_sem[b])

    for b in range(_NBUF):
        last_off = base_off + (num_chunks - _NBUF + b) * _CHUNK
        pltpu.make_async_copy(
            out_b[b], out_hbm.at[pl.ds(last_off, _CHUNK)], out_sem[b]).wait()


def kernel(scores, thresholds):
    n = scores.shape[0]
    n_thr = thresholds.shape[0]
    assert n % (_NUM_WORKERS * _CHUNK * _NBUF) == 0
    per_worker = n // _NUM_WORKERS

    thr_b = jnp.repeat(thresholds.astype(jnp.float32), _LANES)

    mesh = plsc.VectorSubcoreMesh(core_axis_name="c", subcore_axis_name="s")
    fn = functools.partial(
        pl.kernel,
        out_type=jax.ShapeDtypeStruct((n,), jnp.int32),
        mesh=mesh,
        compiler_params=pltpu.CompilerParams(needs_layout_passes=False),
        scratch_types=[
            pltpu.VMEM((n_thr * _LANES,), jnp.float32),
            pltpu.VMEM((_CHUNK,), jnp.float32),
            pltpu.VMEM((_CHUNK,), jnp.float32),
            pltpu.VMEM((_CHUNK,), jnp.int32),
            pltpu.VMEM((_CHUNK,), jnp.int32),
            pltpu.SemaphoreType.DMA,
            pltpu.SemaphoreType.DMA,
            pltpu.SemaphoreType.DMA,
            pltpu.SemaphoreType.DMA,
        ],
    )(functools.partial(_sc_kernel_body, n_thr, per_worker))
    return fn(scores, thr_b)
